# baseline (speedup 1.0000x reference)
import functools

import jax
import jax.numpy as jnp
from jax import lax
from jax.experimental import pallas as pl
from jax.experimental.pallas import tpu as pltpu
from jax.experimental.pallas import tpu_sc as plsc

N = 10000
E = 320000
G = 64
DIN = 128
EMB = 128
NUM_LAYER = 5

NC = 2
NS = 16
NW = NC * NS
EPW = E // NW
CHUNK = 40
NBUF = 5
NSUP = EPW // (NBUF * CHUNK)
ROWS = 624
TAIL = N - NS * ROWS

BM = 10000


def _seg_sum_kernel():
    mesh = plsc.VectorSubcoreMesh(core_axis_name="c", subcore_axis_name="s")

    @functools.partial(
        pl.kernel,
        out_type=jax.ShapeDtypeStruct((NC, N, EMB), jnp.float32),
        mesh=mesh,
        scratch_types=[
            pltpu.VMEM_SHARED((N, EMB), jnp.float32),
            [pltpu.VMEM((2, NBUF, CHUNK), jnp.int32) for _ in range(2)],
            [pltpu.VMEM((2, NBUF, CHUNK), jnp.int32) for _ in range(2)],
            [pltpu.VMEM((CHUNK, EMB), jnp.float32) for _ in range(NBUF)],
            [pltpu.SemaphoreType.DMA for _ in range(NBUF)],
            pltpu.SemaphoreType.DMA,
            pltpu.SemaphoreType.DMA,
        ],
    )
    def seg(h_hbm, src_hbm, dst_hbm, zeros_hbm, out_hbm,
            agg_sh, sidx, didx, rows_v, gsem, ssem, isem):
        c = lax.axis_index("c")
        s = lax.axis_index("s")
        wid = c * NS + s
        pltpu.sync_copy(src_hbm.at[wid, pl.ds(0, 2)], sidx[0])
        pltpu.sync_copy(dst_hbm.at[wid, pl.ds(0, 2)], didx[0])
        pltpu.async_copy(src_hbm.at[wid, pl.ds(2, 2)], sidx[1], isem)
        pltpu.async_copy(dst_hbm.at[wid, pl.ds(2, 2)], didx[1], isem)

        for b in range(NBUF):
            pltpu.async_copy(h_hbm.at[sidx[0].at[0, b]], rows_v[b], gsem[b])

        pltpu.sync_copy(zeros_hbm.at[pl.ds(s * ROWS, ROWS)],
                        agg_sh.at[pl.ds(s * ROWS, ROWS)])

        @pl.when(s == NS - 1)
        def _():
            pltpu.sync_copy(zeros_hbm.at[pl.ds(NS * ROWS, TAIL)],
                            agg_sh.at[pl.ds(NS * ROWS, TAIL)])

        plsc.subcore_barrier()

        def load_pair(p0, bank):
            pltpu.async_copy(src_hbm.at[wid, pl.ds(p0, 2)], sidx[bank], isem)
            pltpu.async_copy(dst_hbm.at[wid, pl.ds(p0, 2)], didx[bank], isem)

        def wait_pair(p0, bank):
            pltpu.make_async_copy(src_hbm.at[wid, pl.ds(p0, 2)],
                                  sidx[bank], isem).wait()
            pltpu.make_async_copy(dst_hbm.at[wid, pl.ds(p0, 2)],
                                  didx[bank], isem).wait()

        def wait_scatter(pqs, psb2, pb):
            pltpu.make_async_copy(
                rows_v[pb], agg_sh.at[didx[pqs].at[psb2, pb]], ssem).wait()

        def superstep(g, qs, sb2, *, first=False, guard_tail=False):
            if sb2 == 1:
                if guard_tail:
                    @pl.when(g + 1 < NSUP)
                    def _():
                        wait_pair(g + 1, 1 - qs)
                else:
                    wait_pair(g + 1, 1 - qs)

            for b in range(NBUF):
                pltpu.make_async_copy(h_hbm.at[sidx[qs].at[sb2, b]],
                                      rows_v[b], gsem[b]).wait()
                if first and b == 0:
                    pass
                elif b == 0:
                    wait_scatter(qs if sb2 == 1 else 1 - qs, 1 - sb2, NBUF - 1)
                else:
                    wait_scatter(qs, sb2, b - 1)
                if sb2 == 0 and b == 0 and not first:
                    @pl.when(g + 2 < NSUP)
                    def _():
                        load_pair(g + 2, 1 - qs)
                pltpu.async_copy(rows_v[b], agg_sh.at[didx[qs].at[sb2, b]],
                                 ssem, add=True)
                if b == 0:
                    if not first:
                        pltpu.async_copy(h_hbm.at[sidx[qs].at[sb2, NBUF - 1]],
                                         rows_v[NBUF - 1], gsem[NBUF - 1])
                elif sb2 == 0:
                    pltpu.async_copy(h_hbm.at[sidx[qs].at[1, b - 1]],
                                     rows_v[b - 1], gsem[b - 1])
                else:
                    if guard_tail:
                        @pl.when(g + 1 < NSUP)
                        def _():
                            pltpu.async_copy(
                                h_hbm.at[sidx[1 - qs].at[0, b - 1]],
                                rows_v[b - 1], gsem[b - 1])
                    else:
                        pltpu.async_copy(h_hbm.at[sidx[1 - qs].at[0, b - 1]],
                                         rows_v[b - 1], gsem[b - 1])

        superstep(0, 0, 0, first=True)
        superstep(1, 0, 1)

        def body(k, carry):
            g0 = 4 * k + 2
            superstep(g0, 1, 0)
            superstep(g0 + 1, 1, 1)
            superstep(g0 + 2, 0, 0)
            superstep(g0 + 3, 0, 1, guard_tail=True)
            return carry

        lax.fori_loop(0, (NSUP - 2) // 4, body, 0)
        wait_scatter(0, 1, NBUF - 1)
        plsc.subcore_barrier()
        pltpu.sync_copy(agg_sh.at[pl.ds(s * ROWS, ROWS)],
                        out_hbm.at[c, pl.ds(s * ROWS, ROWS)])

        @pl.when(s == NS - 1)
        def _():
            pltpu.sync_copy(agg_sh.at[pl.ds(NS * ROWS, TAIL)],
                            out_hbm.at[c, pl.ds(NS * ROWS, TAIL)])

    return seg


_seg_sum = _seg_sum_kernel()


def _proj_body(x_ref, w_ref, b_ref, o_ref):
    o_ref[...] = (
        jnp.dot(x_ref[...], w_ref[...], preferred_element_type=jnp.float32)
        + b_ref[...]
    )


def _proj(x, w, b):
    return pl.pallas_call(
        _proj_body,
        grid=(N // BM,),
        in_specs=[
            pl.BlockSpec((BM, DIN), lambda i: (i, 0)),
            pl.BlockSpec((DIN, EMB), lambda i: (0, 0)),
            pl.BlockSpec((1, EMB), lambda i: (0, 0)),
        ],
        out_specs=pl.BlockSpec((BM, EMB), lambda i: (i, 0)),
        out_shape=jax.ShapeDtypeStruct((N, EMB), jnp.float32),
    )(x, w, b.reshape(1, EMB))


def _layer_body(h_ref, a_ref, w1_ref, b1_ref, w2_ref, b2_ref, o_ref):
    z = h_ref[...] + a_ref[0] + a_ref[1]
    t = jnp.maximum(
        jnp.dot(z, w1_ref[...], preferred_element_type=jnp.float32)
        + b1_ref[...],
        0.0,
    )
    hn = (
        jnp.dot(t, w2_ref[...], preferred_element_type=jnp.float32)
        + b2_ref[...]
    )
    o_ref[...] = jnp.maximum(hn, 0.0)


def _layer(h, agg2, w1, b1, w2, b2):
    return pl.pallas_call(
        _layer_body,
        grid=(N // BM,),
        in_specs=[
            pl.BlockSpec((BM, EMB), lambda i: (i, 0)),
            pl.BlockSpec((NC, BM, EMB), lambda i: (0, i, 0)),
            pl.BlockSpec((EMB, 2 * EMB), lambda i: (0, 0)),
            pl.BlockSpec((1, 2 * EMB), lambda i: (0, 0)),
            pl.BlockSpec((2 * EMB, EMB), lambda i: (0, 0)),
            pl.BlockSpec((1, EMB), lambda i: (0, 0)),
        ],
        out_specs=pl.BlockSpec((BM, EMB), lambda i: (i, 0)),
        out_shape=jax.ShapeDtypeStruct((N, EMB), jnp.float32),
    )(h, agg2, w1, b1.reshape(1, 2 * EMB), w2, b2.reshape(1, EMB))


def _tail_body(h_ref, a_ref, w1_ref, b1_ref, w2_ref, b2_ref, bat_ref,
               wg1_ref, bg1_ref, wg2_ref, bg2_ref,
               wp1_ref, bp1_ref, wp2_ref, bp2_ref,
               o1_ref, o2_ref, acc1, acc2):
    i = pl.program_id(0)

    @pl.when(i == 0)
    def _():
        acc1[...] = jnp.zeros_like(acc1)
        acc2[...] = jnp.zeros_like(acc2)

    z = h_ref[...] + a_ref[0] + a_ref[1]
    t = jnp.maximum(
        jnp.dot(z, w1_ref[...], preferred_element_type=jnp.float32)
        + b1_ref[...],
        0.0,
    )
    h = (
        jnp.dot(t, w2_ref[...], preferred_element_type=jnp.float32)
        + b2_ref[...]
    )
    g1 = jax.nn.sigmoid(
        jnp.dot(h, wg1_ref[...], preferred_element_type=jnp.float32)
        + bg1_ref[...]
    )
    g2 = jax.nn.sigmoid(
        jnp.dot(h, wg2_ref[...], preferred_element_type=jnp.float32)
        + bg2_ref[...]
    )
    bat = bat_ref[0]
    onehot_t = (
        bat == lax.broadcasted_iota(jnp.int32, (G, BM), 0)
    ).astype(jnp.float32)
    acc1[...] += lax.dot_general(
        onehot_t, h * g1, (((1,), (0,)), ((), ())),
        preferred_element_type=jnp.float32,
    )
    acc2[...] += lax.dot_general(
        onehot_t, h * g2, (((1,), (0,)), ((), ())),
        preferred_element_type=jnp.float32,
    )

    @pl.when(i == pl.num_programs(0) - 1)
    def _():
        o1_ref[...] = (
            jnp.dot(acc1[...], wp1_ref[...], preferred_element_type=jnp.float32)
            + bp1_ref[...]
        )
        o2_ref[...] = (
            jnp.dot(acc2[...], wp2_ref[...], preferred_element_type=jnp.float32)
            + bp2_ref[...]
        )


def _tail(h, agg2, w1, b1, w2, b2, batch_i32, p):
    bat3 = batch_i32.reshape(N // BM, 1, BM)
    wcol = pl.BlockSpec((EMB, EMB), lambda i: (0, 0))
    bcol = pl.BlockSpec((1, EMB), lambda i: (0, 0))
    return pl.pallas_call(
        _tail_body,
        grid=(N // BM,),
        in_specs=[
            pl.BlockSpec((BM, EMB), lambda i: (i, 0)),
            pl.BlockSpec((NC, BM, EMB), lambda i: (0, i, 0)),
            pl.BlockSpec((EMB, 2 * EMB), lambda i: (0, 0)),
            pl.BlockSpec((1, 2 * EMB), lambda i: (0, 0)),
            pl.BlockSpec((2 * EMB, EMB), lambda i: (0, 0)),
            bcol,
            pl.BlockSpec((1, 1, BM), lambda i: (i, 0, 0)),
            wcol, bcol, wcol, bcol, wcol, bcol, wcol, bcol,
        ],
        out_specs=[
            pl.BlockSpec((G, EMB), lambda i: (0, 0)),
            pl.BlockSpec((G, EMB), lambda i: (0, 0)),
        ],
        out_shape=[
            jax.ShapeDtypeStruct((G, EMB), jnp.float32),
            jax.ShapeDtypeStruct((G, EMB), jnp.float32),
        ],
        scratch_shapes=[
            pltpu.VMEM((G, EMB), jnp.float32),
            pltpu.VMEM((G, EMB), jnp.float32),
        ],
    )(h, agg2, w1, b1.reshape(1, 2 * EMB), w2, b2.reshape(1, EMB), bat3,
      p['Wg1'], p['bg1'].reshape(1, EMB), p['Wg2'], p['bg2'].reshape(1, EMB),
      p['Wp1'], p['bp1'].reshape(1, EMB), p['Wp2'], p['bp2'].reshape(1, EMB))


def kernel(x, params, edge_index, batch):
    src = edge_index[0].astype(jnp.int32).reshape(NW, NSUP, NBUF, CHUNK)
    dst = edge_index[1].astype(jnp.int32).reshape(NW, NSUP, NBUF, CHUNK)
    batch_i32 = batch.astype(jnp.int32)
    zeros = jnp.zeros((N, EMB), jnp.float32)

    h = _proj(x, params['W_in'], params['b_in'])
    for l in range(NUM_LAYER - 1):
        lp = params['layers'][l]
        agg2 = _seg_sum(h, src, dst, zeros)
        h = _layer(h, agg2, lp['W1'], lp['b1'], lp['W2'], lp['b2'])
    lp = params['layers'][NUM_LAYER - 1]
    agg2 = _seg_sum(h, src, dst, zeros)
    out1, out2 = _tail(h, agg2, lp['W1'], lp['b1'], lp['W2'], lp['b2'],
                       batch_i32, params)
    return (out1, out2)


# speedup vs baseline: 1.0201x; 1.0201x over previous
import functools

import jax
import jax.numpy as jnp
from jax import lax
from jax.experimental import pallas as pl
from jax.experimental.pallas import tpu as pltpu
from jax.experimental.pallas import tpu_sc as plsc

N = 10000
E = 320000
G = 64
DIN = 128
EMB = 128
NUM_LAYER = 5

NC = 2
NS = 16
NW = NC * NS
EPW = E // NW
CHUNK = 40
NBUF = 5
NSUP = EPW // (NBUF * CHUNK)
ROWS = 624
TAIL = N - NS * ROWS

BM = 5000


def _seg_sum_kernel():
    mesh = plsc.VectorSubcoreMesh(core_axis_name="c", subcore_axis_name="s")

    @functools.partial(
        pl.kernel,
        out_type=jax.ShapeDtypeStruct((NC, N, EMB), jnp.float32),
        mesh=mesh,
        scratch_types=[
            pltpu.VMEM_SHARED((N, EMB), jnp.float32),
            [pltpu.VMEM((2, NBUF, CHUNK), jnp.int32) for _ in range(2)],
            [pltpu.VMEM((2, NBUF, CHUNK), jnp.int32) for _ in range(2)],
            [pltpu.VMEM((CHUNK, EMB), jnp.float32) for _ in range(NBUF)],
            [pltpu.SemaphoreType.DMA for _ in range(NBUF)],
            pltpu.SemaphoreType.DMA,
            pltpu.SemaphoreType.DMA,
        ],
    )
    def seg(h_hbm, src_hbm, dst_hbm, zeros_hbm, out_hbm,
            agg_sh, sidx, didx, rows_v, gsem, ssem, isem):
        c = lax.axis_index("c")
        s = lax.axis_index("s")
        wid = c * NS + s
        pltpu.sync_copy(src_hbm.at[wid, pl.ds(0, 2)], sidx[0])
        pltpu.sync_copy(dst_hbm.at[wid, pl.ds(0, 2)], didx[0])
        pltpu.async_copy(src_hbm.at[wid, pl.ds(2, 2)], sidx[1], isem)
        pltpu.async_copy(dst_hbm.at[wid, pl.ds(2, 2)], didx[1], isem)

        for b in range(NBUF):
            pltpu.async_copy(h_hbm.at[sidx[0].at[0, b]], rows_v[b], gsem[b])

        pltpu.sync_copy(zeros_hbm.at[pl.ds(s * ROWS, ROWS)],
                        agg_sh.at[pl.ds(s * ROWS, ROWS)])

        @pl.when(s == NS - 1)
        def _():
            pltpu.sync_copy(zeros_hbm.at[pl.ds(NS * ROWS, TAIL)],
                            agg_sh.at[pl.ds(NS * ROWS, TAIL)])

        plsc.subcore_barrier()

        def load_pair(p0, bank):
            pltpu.async_copy(src_hbm.at[wid, pl.ds(p0, 2)], sidx[bank], isem)
            pltpu.async_copy(dst_hbm.at[wid, pl.ds(p0, 2)], didx[bank], isem)

        def wait_pair(p0, bank):
            pltpu.make_async_copy(src_hbm.at[wid, pl.ds(p0, 2)],
                                  sidx[bank], isem).wait()
            pltpu.make_async_copy(dst_hbm.at[wid, pl.ds(p0, 2)],
                                  didx[bank], isem).wait()

        def wait_scatter(pqs, psb2, pb):
            pltpu.make_async_copy(
                rows_v[pb], agg_sh.at[didx[pqs].at[psb2, pb]], ssem).wait()

        def superstep(g, qs, sb2, *, first=False, guard_tail=False):
            if sb2 == 1:
                if guard_tail:
                    @pl.when(g + 1 < NSUP)
                    def _():
                        wait_pair(g + 1, 1 - qs)
                else:
                    wait_pair(g + 1, 1 - qs)

            for b in range(NBUF):
                pltpu.make_async_copy(h_hbm.at[sidx[qs].at[sb2, b]],
                                      rows_v[b], gsem[b]).wait()
                if first and b == 0:
                    pass
                elif b == 0:
                    wait_scatter(qs if sb2 == 1 else 1 - qs, 1 - sb2, NBUF - 1)
                else:
                    wait_scatter(qs, sb2, b - 1)
                if sb2 == 0 and b == 0 and not first:
                    @pl.when(g + 2 < NSUP)
                    def _():
                        load_pair(g + 2, 1 - qs)
                pltpu.async_copy(rows_v[b], agg_sh.at[didx[qs].at[sb2, b]],
                                 ssem, add=True)
                if b == 0:
                    if not first:
                        pltpu.async_copy(h_hbm.at[sidx[qs].at[sb2, NBUF - 1]],
                                         rows_v[NBUF - 1], gsem[NBUF - 1])
                elif sb2 == 0:
                    pltpu.async_copy(h_hbm.at[sidx[qs].at[1, b - 1]],
                                     rows_v[b - 1], gsem[b - 1])
                else:
                    if guard_tail:
                        @pl.when(g + 1 < NSUP)
                        def _():
                            pltpu.async_copy(
                                h_hbm.at[sidx[1 - qs].at[0, b - 1]],
                                rows_v[b - 1], gsem[b - 1])
                    else:
                        pltpu.async_copy(h_hbm.at[sidx[1 - qs].at[0, b - 1]],
                                         rows_v[b - 1], gsem[b - 1])

        superstep(0, 0, 0, first=True)
        superstep(1, 0, 1)

        def body(k, carry):
            g0 = 4 * k + 2
            superstep(g0, 1, 0)
            superstep(g0 + 1, 1, 1)
            superstep(g0 + 2, 0, 0)
            superstep(g0 + 3, 0, 1, guard_tail=True)
            return carry

        lax.fori_loop(0, (NSUP - 2) // 4, body, 0)
        wait_scatter(0, 1, NBUF - 1)
        plsc.subcore_barrier()
        pltpu.sync_copy(agg_sh.at[pl.ds(s * ROWS, ROWS)],
                        out_hbm.at[c, pl.ds(s * ROWS, ROWS)])

        @pl.when(s == NS - 1)
        def _():
            pltpu.sync_copy(agg_sh.at[pl.ds(NS * ROWS, TAIL)],
                            out_hbm.at[c, pl.ds(NS * ROWS, TAIL)])

    return seg


_seg_sum = _seg_sum_kernel()


def _proj_body(x_ref, w_ref, b_ref, o_ref):
    o_ref[...] = (
        jnp.dot(x_ref[...], w_ref[...], preferred_element_type=jnp.float32)
        + b_ref[...]
    )


def _proj(x, w, b):
    return pl.pallas_call(
        _proj_body,
        grid=(N // BM,),
        in_specs=[
            pl.BlockSpec((BM, DIN), lambda i: (i, 0)),
            pl.BlockSpec((DIN, EMB), lambda i: (0, 0)),
            pl.BlockSpec((1, EMB), lambda i: (0, 0)),
        ],
        out_specs=pl.BlockSpec((BM, EMB), lambda i: (i, 0)),
        out_shape=jax.ShapeDtypeStruct((N, EMB), jnp.float32),
    )(x, w, b.reshape(1, EMB))


def _layer_body(h_ref, a_ref, w1_ref, b1_ref, w2_ref, b2_ref, o_ref):
    z = h_ref[...] + a_ref[0] + a_ref[1]
    t = jnp.maximum(
        jnp.dot(z, w1_ref[...], preferred_element_type=jnp.float32)
        + b1_ref[...],
        0.0,
    )
    hn = (
        jnp.dot(t, w2_ref[...], preferred_element_type=jnp.float32)
        + b2_ref[...]
    )
    o_ref[...] = jnp.maximum(hn, 0.0)


def _layer(h, agg2, w1, b1, w2, b2):
    return pl.pallas_call(
        _layer_body,
        grid=(N // BM,),
        in_specs=[
            pl.BlockSpec((BM, EMB), lambda i: (i, 0)),
            pl.BlockSpec((NC, BM, EMB), lambda i: (0, i, 0)),
            pl.BlockSpec((EMB, 2 * EMB), lambda i: (0, 0)),
            pl.BlockSpec((1, 2 * EMB), lambda i: (0, 0)),
            pl.BlockSpec((2 * EMB, EMB), lambda i: (0, 0)),
            pl.BlockSpec((1, EMB), lambda i: (0, 0)),
        ],
        out_specs=pl.BlockSpec((BM, EMB), lambda i: (i, 0)),
        out_shape=jax.ShapeDtypeStruct((N, EMB), jnp.float32),
    )(h, agg2, w1, b1.reshape(1, 2 * EMB), w2, b2.reshape(1, EMB))


def _tail_body(h_ref, a_ref, w1_ref, b1_ref, w2_ref, b2_ref, bat_ref,
               wg1_ref, bg1_ref, wg2_ref, bg2_ref,
               wp1_ref, bp1_ref, wp2_ref, bp2_ref,
               o1_ref, o2_ref, acc1, acc2):
    i = pl.program_id(0)

    @pl.when(i == 0)
    def _():
        acc1[...] = jnp.zeros_like(acc1)
        acc2[...] = jnp.zeros_like(acc2)

    z = h_ref[...] + a_ref[0] + a_ref[1]
    t = jnp.maximum(
        jnp.dot(z, w1_ref[...], preferred_element_type=jnp.float32)
        + b1_ref[...],
        0.0,
    )
    h = (
        jnp.dot(t, w2_ref[...], preferred_element_type=jnp.float32)
        + b2_ref[...]
    )
    g1 = jax.nn.sigmoid(
        jnp.dot(h, wg1_ref[...], preferred_element_type=jnp.float32)
        + bg1_ref[...]
    )
    g2 = jax.nn.sigmoid(
        jnp.dot(h, wg2_ref[...], preferred_element_type=jnp.float32)
        + bg2_ref[...]
    )
    bat = bat_ref[0]
    onehot_t = (
        bat == lax.broadcasted_iota(jnp.int32, (G, BM), 0)
    ).astype(jnp.float32)
    acc1[...] += lax.dot_general(
        onehot_t, h * g1, (((1,), (0,)), ((), ())),
        preferred_element_type=jnp.float32,
    )
    acc2[...] += lax.dot_general(
        onehot_t, h * g2, (((1,), (0,)), ((), ())),
        preferred_element_type=jnp.float32,
    )

    @pl.when(i == pl.num_programs(0) - 1)
    def _():
        o1_ref[...] = (
            jnp.dot(acc1[...], wp1_ref[...], preferred_element_type=jnp.float32)
            + bp1_ref[...]
        )
        o2_ref[...] = (
            jnp.dot(acc2[...], wp2_ref[...], preferred_element_type=jnp.float32)
            + bp2_ref[...]
        )


def _tail(h, agg2, w1, b1, w2, b2, batch_i32, p):
    bat3 = batch_i32.reshape(N // BM, 1, BM)
    wcol = pl.BlockSpec((EMB, EMB), lambda i: (0, 0))
    bcol = pl.BlockSpec((1, EMB), lambda i: (0, 0))
    return pl.pallas_call(
        _tail_body,
        grid=(N // BM,),
        in_specs=[
            pl.BlockSpec((BM, EMB), lambda i: (i, 0)),
            pl.BlockSpec((NC, BM, EMB), lambda i: (0, i, 0)),
            pl.BlockSpec((EMB, 2 * EMB), lambda i: (0, 0)),
            pl.BlockSpec((1, 2 * EMB), lambda i: (0, 0)),
            pl.BlockSpec((2 * EMB, EMB), lambda i: (0, 0)),
            bcol,
            pl.BlockSpec((1, 1, BM), lambda i: (i, 0, 0)),
            wcol, bcol, wcol, bcol, wcol, bcol, wcol, bcol,
        ],
        out_specs=[
            pl.BlockSpec((G, EMB), lambda i: (0, 0)),
            pl.BlockSpec((G, EMB), lambda i: (0, 0)),
        ],
        out_shape=[
            jax.ShapeDtypeStruct((G, EMB), jnp.float32),
            jax.ShapeDtypeStruct((G, EMB), jnp.float32),
        ],
        scratch_shapes=[
            pltpu.VMEM((G, EMB), jnp.float32),
            pltpu.VMEM((G, EMB), jnp.float32),
        ],
    )(h, agg2, w1, b1.reshape(1, 2 * EMB), w2, b2.reshape(1, EMB), bat3,
      p['Wg1'], p['bg1'].reshape(1, EMB), p['Wg2'], p['bg2'].reshape(1, EMB),
      p['Wp1'], p['bp1'].reshape(1, EMB), p['Wp2'], p['bp2'].reshape(1, EMB))


def kernel(x, params, edge_index, batch):
    src = edge_index[0].astype(jnp.int32).reshape(NW, NSUP, NBUF, CHUNK)
    dst = edge_index[1].astype(jnp.int32).reshape(NW, NSUP, NBUF, CHUNK)
    batch_i32 = batch.astype(jnp.int32)
    zeros = jnp.zeros((N, EMB), jnp.float32)

    h = _proj(x, params['W_in'], params['b_in'])
    for l in range(NUM_LAYER - 1):
        lp = params['layers'][l]
        agg2 = _seg_sum(h, src, dst, zeros)
        h = _layer(h, agg2, lp['W1'], lp['b1'], lp['W2'], lp['b2'])
    lp = params['layers'][NUM_LAYER - 1]
    agg2 = _seg_sum(h, src, dst, zeros)
    out1, out2 = _tail(h, agg2, lp['W1'], lp['b1'], lp['W2'], lp['b2'],
                       batch_i32, params)
    return (out1, out2)
